# baseline (speedup 1.0000x reference)
import functools

import jax
import jax.numpy as jnp
from jax import lax
from jax.experimental import pallas as pl
from jax.experimental.pallas import tpu as pltpu
from jax.experimental.pallas import tpu_sc as plsc

BATCH = 16384
D = 32
NC = 2
NS = 16
L = 16
NW = NC * NS
BPW = BATCH // NW

_mesh = plsc.VectorSubcoreMesh(core_axis_name="c", subcore_axis_name="s")


@functools.partial(
    pl.kernel,
    mesh=_mesh,
    compiler_params=pltpu.CompilerParams(
        needs_layout_passes=False, use_tc_tiling_on_sc=False),
    out_type=jax.ShapeDtypeStruct((BATCH,), jnp.float32),
    scratch_types=[
        pltpu.VMEM((BPW,), jnp.int32),
        pltpu.VMEM((BPW,), jnp.int32),
        pltpu.VMEM((BPW, D), jnp.float32),
        pltpu.VMEM((BPW, D), jnp.float32),
        pltpu.VMEM((BPW,), jnp.float32),
        pltpu.SemaphoreType.DMA,
        pltpu.SemaphoreType.DMA,
    ],
)
def _ncf_sc(uidx_hbm, iidx_hbm, utab_hbm, itab_hbm, out_hbm,
            uidx_v, iidx_v, urows_v, irows_v, scores_v, sem_u, sem_i):
    wid = lax.axis_index("s") * NC + lax.axis_index("c")
    base = wid * BPW

    pltpu.sync_copy(uidx_hbm.at[pl.ds(base, BPW)], uidx_v)
    pltpu.sync_copy(iidx_hbm.at[pl.ds(base, BPW)], iidx_v)
    cp_u = pltpu.async_copy(utab_hbm.at[uidx_v], urows_v, sem_u)
    cp_i = pltpu.async_copy(itab_hbm.at[iidx_v], irows_v, sem_i)
    cp_u.wait()
    cp_i.wait()

    lane = lax.iota(jnp.int32, L)

    def body(g, carry):
        acc = jnp.zeros((L,), jnp.float32)
        for k in range(L):
            r = g * L + k
            h = (urows_v[r, pl.ds(0, L)] * irows_v[r, pl.ds(0, L)]
                 + urows_v[r, pl.ds(L, L)] * irows_v[r, pl.ds(L, L)])
            acc = jnp.where(lane == k, jnp.sum(h), acc)
        scores_v[pl.ds(g * L, L)] = acc
        return carry

    lax.fori_loop(0, BPW // L, body, 0)

    pltpu.sync_copy(scores_v, out_hbm.at[pl.ds(base, BPW)])


def kernel(user_idx, item_idx, user_table, item_table):
    return _ncf_sc(user_idx.astype(jnp.int32), item_idx.astype(jnp.int32),
                   user_table, item_table)


# speedup vs baseline: 3.2773x; 3.2773x over previous
import functools

import jax
import jax.numpy as jnp
from jax import lax
from jax.experimental import pallas as pl
from jax.experimental.pallas import tpu as pltpu
from jax.experimental.pallas import tpu_sc as plsc

BATCH = 16384
D = 32
NC = 2
NS = 16
L = 16
NW = NC * NS
BPW = BATCH // NW
K = 16
NCHK = BPW // K

_mesh = plsc.VectorSubcoreMesh(core_axis_name="c", subcore_axis_name="s")


@functools.partial(
    pl.kernel,
    mesh=_mesh,
    compiler_params=pltpu.CompilerParams(needs_layout_passes=False),
    out_type=jax.ShapeDtypeStruct((BATCH,), jnp.float32),
    scratch_types=[
        pltpu.VMEM((BPW,), jnp.int32),
        pltpu.VMEM((BPW,), jnp.int32),
        pltpu.VMEM((K * D, 128), jnp.float32),
        pltpu.VMEM((D, BPW), jnp.float32),
        pltpu.VMEM((D, BPW), jnp.float32),
        pltpu.VMEM((BPW,), jnp.float32),
        pltpu.SemaphoreType.DMA,
    ],
)
def _ncf_sc(uidx_hbm, iidx_hbm, utab_hbm, itab_hbm, out_hbm,
            uidx_v, iidx_v, dst_v, ucomp_v, icomp_v, scores_v, sem):
    wid = lax.axis_index("s") * NC + lax.axis_index("c")
    base = wid * BPW

    pltpu.sync_copy(uidx_hbm.at[pl.ds(base, BPW)], uidx_v)
    pltpu.sync_copy(iidx_hbm.at[pl.ds(base, BPW)], iidx_v)

    lane = lax.iota(jnp.int32, L)

    def make_phase(idx_v, tab_hbm, comp_v):
        def chunk(c, carry):
            vec = idx_v[pl.ds(c * K, K)]
            cps = []
            for k in range(K):
                ii = jnp.sum(jnp.where(lane == k, vec, 0))
                su = pl.multiple_of((ii >> 7) * 128, 128)
                cps.append(pltpu.async_copy(
                    tab_hbm.at[:, pl.ds(su, 128)],
                    dst_v.at[pl.ds(k * D, D), :], sem))
            for cp in cps:
                cp.wait()
            for k in range(K):
                ii = jnp.sum(jnp.where(lane == k, vec, 0))
                lu = ii & 127
                p = c * K + k
                for h in range(2):
                    rows = k * D + h * L + lane
                    vals = plsc.load_gather(
                        dst_v, [rows, jnp.full((L,), 0, jnp.int32) + lu])
                    plsc.store_scatter(
                        comp_v, [h * L + lane, jnp.full((L,), 0, jnp.int32) + p],
                        vals)
            return carry
        return chunk

    lax.fori_loop(0, NCHK, make_phase(uidx_v, utab_hbm, ucomp_v), 0)
    lax.fori_loop(0, NCHK, make_phase(iidx_v, itab_hbm, icomp_v), 0)

    def dot(p, carry):
        acc = jnp.zeros((L,), jnp.float32)
        for j in range(D):
            acc = acc + (ucomp_v[j, pl.ds(p * L, L)]
                         * icomp_v[j, pl.ds(p * L, L)])
        scores_v[pl.ds(p * L, L)] = acc
        return carry

    lax.fori_loop(0, BPW // L, dot, 0)

    pltpu.sync_copy(scores_v, out_hbm.at[pl.ds(base, BPW)])


def kernel(user_idx, item_idx, user_table, item_table):
    return _ncf_sc(user_idx.astype(jnp.int32), item_idx.astype(jnp.int32),
                   user_table.T, item_table.T)
